# Initial kernel scaffold; baseline (speedup 1.0000x reference)
import functools

import jax
import jax.numpy as jnp
from jax import lax
from jax.experimental import pallas as pl
from jax.experimental.pallas import tpu as pltpu
from jax.experimental.pallas import tpu_sc as plsc

N = 10000
E = 320000
D = 128

NC = 2
NS = 16
NW = NC * NS

CB = 125
SUBS = E // CB
PER_W = SUBS // NW
RPT = N // NS


def _sc_body(h_hbm, src_hbm, dst_hbm, ea_hbm, out_hbm,
             sidx, didx, gbuf, ebuf, acc, sem):
    c = lax.axis_index("c")
    s = lax.axis_index("s")
    wid = s * NC + c

    zero = jnp.zeros((16,), jnp.float32)

    def zrow(r, carry):
        for k in range(D // 16):
            gbuf[r, pl.ds(k * 16, 16)] = zero
        return carry

    lax.fori_loop(0, CB, zrow, 0)
    for t in range(RPT // CB):
        pltpu.sync_copy(gbuf, acc.at[pl.ds(s * RPT + t * CB, CB)])
    plsc.subcore_barrier()

    def chunk(j, carry):
        jg = wid * PER_W + j
        pltpu.sync_copy(src_hbm.at[jg], sidx)
        pltpu.sync_copy(dst_hbm.at[jg], didx)
        pltpu.sync_copy(ea_hbm.at[jg], ebuf)
        pltpu.async_copy(h_hbm.at[sidx], gbuf, sem).wait()

        def mrow(r, inner):
            for k in range(D // 16):
                sl = pl.ds(k * 16, 16)
                gbuf[r, sl] = jnp.maximum(gbuf[r, sl] + ebuf[r, sl], 0.0)
            return inner

        lax.fori_loop(0, CB, mrow, 0)
        pltpu.sync_copy(gbuf, acc.at[didx], add=True)
        return carry

    lax.fori_loop(0, PER_W, chunk, 0)
    plsc.subcore_barrier()

    pltpu.sync_copy(acc.at[pl.ds(s * RPT, RPT)],
                    out_hbm.at[c, pl.ds(s * RPT, RPT)])


def _make_sc_aggregate():
    mesh = plsc.VectorSubcoreMesh(core_axis_name="c", subcore_axis_name="s",
                                  num_cores=NC, num_subcores=NS)
    return pl.kernel(
        _sc_body,
        out_type=jax.ShapeDtypeStruct((NC, N, D), jnp.float32),
        mesh=mesh,
        scratch_types=[
            pltpu.VMEM((CB,), jnp.int32),
            pltpu.VMEM((CB,), jnp.int32),
            pltpu.VMEM((CB, D), jnp.float32),
            pltpu.VMEM((CB, D), jnp.float32),
            pltpu.VMEM_SHARED((N, D), jnp.float32),
            pltpu.SemaphoreType.DMA,
        ],
    )


def _mlp_body(h_ref, a0_ref, a1_ref, w1_ref, b1_ref, w2_ref, b2_ref, o_ref):
    t = h_ref[...] + a0_ref[...] + a1_ref[...]
    u = jnp.maximum(
        jnp.dot(t, w1_ref[...], preferred_element_type=jnp.float32)
        + b1_ref[...], 0.0)
    v = jnp.maximum(
        jnp.dot(u, w2_ref[...], preferred_element_type=jnp.float32)
        + b2_ref[...], 0.0)
    o_ref[...] = v


def _jk_body(h1_ref, h2_ref, h3_ref, wc1_ref, wc2_ref, wc3_ref, bc_ref, o_ref):
    acc = jnp.dot(h1_ref[...], wc1_ref[...], preferred_element_type=jnp.float32)
    acc += jnp.dot(h2_ref[...], wc2_ref[...], preferred_element_type=jnp.float32)
    acc += jnp.dot(h3_ref[...], wc3_ref[...], preferred_element_type=jnp.float32)
    o_ref[...] = jnp.maximum(acc + bc_ref[...], 0.0)


_ROWS = 1000
_GRID = N // _ROWS


def _row_spec():
    return pl.BlockSpec((_ROWS, D), lambda i: (i, 0))


def _full_spec():
    return pl.BlockSpec((D, D), lambda i: (0, 0))


def _bias_spec():
    return pl.BlockSpec((1, D), lambda i: (0, 0))


def _mlp(h, a0, a1, w1, b1, w2, b2):
    return pl.pallas_call(
        _mlp_body,
        grid=(_GRID,),
        in_specs=[_row_spec(), _row_spec(), _row_spec(),
                  _full_spec(), _bias_spec(), _full_spec(), _bias_spec()],
        out_specs=_row_spec(),
        out_shape=jax.ShapeDtypeStruct((N, D), jnp.float32),
    )(h, a0, a1, w1, b1.reshape(1, D), w2, b2.reshape(1, D))


def _jk(h1, h2, h3, wc, bc):
    return pl.pallas_call(
        _jk_body,
        grid=(_GRID,),
        in_specs=[_row_spec(), _row_spec(), _row_spec(),
                  _full_spec(), _full_spec(), _full_spec(), _bias_spec()],
        out_specs=_row_spec(),
        out_shape=jax.ShapeDtypeStruct((N, D), jnp.float32),
    )(h1, h2, h3, wc[:D], wc[D:2 * D], wc[2 * D:], bc.reshape(1, D))


@jax.jit
def kernel(x, edge_index, edge_attr,
           W1_0, b1_0, W2_0, b2_0,
           W1_1, b1_1, W2_1, b2_1,
           W1_2, b1_2, W2_2, b2_2,
           Wc, bc):
    src = edge_index[0].astype(jnp.int32).reshape(SUBS, CB)
    dst = edge_index[1].astype(jnp.int32).reshape(SUBS, CB)
    ea = edge_attr.reshape(SUBS, CB, D)

    aggregate = _make_sc_aggregate()

    params = [(W1_0, b1_0, W2_0, b2_0),
              (W1_1, b1_1, W2_1, b2_1),
              (W1_2, b1_2, W2_2, b2_2)]
    h = x
    xs = []
    for (w1, b1, w2, b2) in params:
        agg = aggregate(h, src, dst, ea)
        h = _mlp(h, agg[0], agg[1], w1, b1, w2, b2)
        xs.append(h)
    return _jk(xs[0], xs[1], xs[2], Wc, bc)


# speedup vs baseline: 3.2344x; 3.2344x over previous
import functools

import jax
import jax.numpy as jnp
from jax import lax
from jax.experimental import pallas as pl
from jax.experimental.pallas import tpu as pltpu
from jax.experimental.pallas import tpu_sc as plsc

N = 10000
E = 320000
D = 128

NC = 2
NS = 16
NW = NC * NS

CB = 125
SUBS = E // CB
PER_W = SUBS // NW
FR = 80
FCHUNKS = N // FR


def _sc_body(h_hbm, src_hbm, dst_hbm, ea_hbm, out_hbm,
             sidx, didx, gbuf, ebuf, acc, sem):
    c = lax.axis_index("c")
    s = lax.axis_index("s")
    wid = s * NC + c

    zero = jnp.zeros((16,), jnp.float32)

    def zrow(r, carry):
        for k in range(D // 16):
            gbuf[r, pl.ds(k * 16, 16)] = zero
        return carry

    lax.fori_loop(0, CB, zrow, 0)
    for i in range(pl.cdiv(FCHUNKS, NS)):
        cid = s + NS * i

        @pl.when(cid < FCHUNKS)
        def _():
            pltpu.sync_copy(gbuf.at[pl.ds(0, FR)],
                            acc.at[pl.ds(cid * FR, FR)])

    plsc.subcore_barrier()

    def chunk(j, carry):
        jg = wid * PER_W + j
        pltpu.sync_copy(src_hbm.at[jg], sidx)
        pltpu.sync_copy(dst_hbm.at[jg], didx)
        pltpu.sync_copy(ea_hbm.at[jg], ebuf)
        pltpu.async_copy(h_hbm.at[sidx], gbuf, sem).wait()

        def mrow(r, inner):
            for k in range(D // 16):
                sl = pl.ds(k * 16, 16)
                gbuf[r, sl] = jnp.maximum(gbuf[r, sl] + ebuf[r, sl], 0.0)
            return inner

        lax.fori_loop(0, CB, mrow, 0)
        pltpu.sync_copy(gbuf, acc.at[didx], add=True)
        return carry

    lax.fori_loop(0, PER_W, chunk, 0)
    plsc.subcore_barrier()

    for i in range(pl.cdiv(FCHUNKS, NS)):
        cid = s + NS * i

        @pl.when(cid < FCHUNKS)
        def _():
            pltpu.sync_copy(acc.at[pl.ds(cid * FR, FR)],
                            out_hbm.at[c, pl.ds(cid * FR, FR)])


def _make_sc_aggregate():
    mesh = plsc.VectorSubcoreMesh(core_axis_name="c", subcore_axis_name="s",
                                  num_cores=NC, num_subcores=NS)
    return pl.kernel(
        _sc_body,
        out_type=jax.ShapeDtypeStruct((NC, N, D), jnp.float32),
        mesh=mesh,
        scratch_types=[
            pltpu.VMEM((CB,), jnp.int32),
            pltpu.VMEM((CB,), jnp.int32),
            pltpu.VMEM((CB, D), jnp.float32),
            pltpu.VMEM((CB, D), jnp.float32),
            pltpu.VMEM_SHARED((N, D), jnp.float32),
            pltpu.SemaphoreType.DMA,
        ],
    )


def _mlp_body(h_ref, a0_ref, a1_ref, w1_ref, b1_ref, w2_ref, b2_ref, o_ref):
    t = h_ref[...] + a0_ref[...] + a1_ref[...]
    u = jnp.maximum(
        jnp.dot(t, w1_ref[...], preferred_element_type=jnp.float32)
        + b1_ref[...], 0.0)
    v = jnp.maximum(
        jnp.dot(u, w2_ref[...], preferred_element_type=jnp.float32)
        + b2_ref[...], 0.0)
    o_ref[...] = v


def _jk_body(h1_ref, h2_ref, h3_ref, wc1_ref, wc2_ref, wc3_ref, bc_ref, o_ref):
    acc = jnp.dot(h1_ref[...], wc1_ref[...], preferred_element_type=jnp.float32)
    acc += jnp.dot(h2_ref[...], wc2_ref[...], preferred_element_type=jnp.float32)
    acc += jnp.dot(h3_ref[...], wc3_ref[...], preferred_element_type=jnp.float32)
    o_ref[...] = jnp.maximum(acc + bc_ref[...], 0.0)


_ROWS = 1000
_GRID = N // _ROWS


def _row_spec():
    return pl.BlockSpec((_ROWS, D), lambda i: (i, 0))


def _full_spec():
    return pl.BlockSpec((D, D), lambda i: (0, 0))


def _bias_spec():
    return pl.BlockSpec((1, D), lambda i: (0, 0))


def _mlp(h, a0, a1, w1, b1, w2, b2):
    return pl.pallas_call(
        _mlp_body,
        grid=(_GRID,),
        in_specs=[_row_spec(), _row_spec(), _row_spec(),
                  _full_spec(), _bias_spec(), _full_spec(), _bias_spec()],
        out_specs=_row_spec(),
        out_shape=jax.ShapeDtypeStruct((N, D), jnp.float32),
    )(h, a0, a1, w1, b1.reshape(1, D), w2, b2.reshape(1, D))


def _jk(h1, h2, h3, wc, bc):
    return pl.pallas_call(
        _jk_body,
        grid=(_GRID,),
        in_specs=[_row_spec(), _row_spec(), _row_spec(),
                  _full_spec(), _full_spec(), _full_spec(), _bias_spec()],
        out_specs=_row_spec(),
        out_shape=jax.ShapeDtypeStruct((N, D), jnp.float32),
    )(h1, h2, h3, wc[:D], wc[D:2 * D], wc[2 * D:], bc.reshape(1, D))


@jax.jit
def kernel(x, edge_index, edge_attr,
           W1_0, b1_0, W2_0, b2_0,
           W1_1, b1_1, W2_1, b2_1,
           W1_2, b1_2, W2_2, b2_2,
           Wc, bc):
    src = edge_index[0].astype(jnp.int32).reshape(SUBS, CB)
    dst = edge_index[1].astype(jnp.int32).reshape(SUBS, CB)
    ea = edge_attr.reshape(SUBS, CB, D)

    aggregate = _make_sc_aggregate()

    params = [(W1_0, b1_0, W2_0, b2_0),
              (W1_1, b1_1, W2_1, b2_1),
              (W1_2, b1_2, W2_2, b2_2)]
    h = x
    xs = []
    for (w1, b1, w2, b2) in params:
        agg = aggregate(h, src, dst, ea)
        h = _mlp(h, agg[0], agg[1], w1, b1, w2, b2)
        xs.append(h)
    return _jk(xs[0], xs[1], xs[2], Wc, bc)


# speedup vs baseline: 6.7023x; 2.0722x over previous
import functools

import jax
import jax.numpy as jnp
from jax import lax
from jax.experimental import pallas as pl
from jax.experimental.pallas import tpu as pltpu
from jax.experimental.pallas import tpu_sc as plsc

N = 10000
E = 320000
D = 128

NC = 2
NS = 16
NW = NC * NS

CB = 80
SUBS = E // CB
PER_W = SUBS // NW
G = 25
NB = PER_W // G
FR = 80
FCHUNKS = N // FR


def _sc_body(h_hbm, src_hbm, dst_hbm, ea_hbm, out_hbm,
             sidx, didx, gbuf0, gbuf1, ebuf0, ebuf1, acc,
             gsem0, gsem1, esem0, esem1):
    c = lax.axis_index("c")
    s = lax.axis_index("s")
    wid = s * NC + c
    gbufs = (gbuf0, gbuf1)
    ebufs = (ebuf0, ebuf1)
    gsems = (gsem0, gsem1)
    esems = (esem0, esem1)

    zero = jnp.zeros((16,), jnp.float32)

    def zrow(r, carry):
        for k in range(D // 16):
            gbuf0[r, pl.ds(k * 16, 16)] = zero
        return carry

    lax.fori_loop(0, FR, zrow, 0)
    for i in range(pl.cdiv(FCHUNKS, NS)):
        cid = s + NS * i

        @pl.when(cid < FCHUNKS)
        def _():
            pltpu.sync_copy(gbuf0.at[pl.ds(0, FR)],
                            acc.at[pl.ds(cid * FR, FR)])

    plsc.subcore_barrier()

    def issue(b, i, p):
        jg = (wid * NB + b) * G + i
        pltpu.async_copy(ea_hbm.at[jg], ebufs[p], esems[p])
        pltpu.async_copy(h_hbm.at[sidx.at[i]], gbufs[p], gsems[p])

    def step(b, i, p):
        gbuf = gbufs[p]
        ebuf = ebufs[p]
        pltpu.make_async_copy(ea_hbm.at[0], ebuf, esems[p]).wait()
        pltpu.make_async_copy(ea_hbm.at[0], gbuf, gsems[p]).wait()

        @pl.when(i + 1 < G)
        def _():
            issue(b, i + 1, 1 - p)

        def mrow(r, inner):
            for k in range(D // 16):
                sl = pl.ds(k * 16, 16)
                gbuf[r, sl] = jnp.maximum(gbuf[r, sl] + ebuf[r, sl], 0.0)
            return inner

        lax.fori_loop(0, CB, mrow, 0)
        pltpu.sync_copy(gbuf, acc.at[didx.at[i]], add=True)

    def block(b, carry):
        pltpu.sync_copy(src_hbm.at[wid, b], sidx)
        pltpu.sync_copy(dst_hbm.at[wid, b], didx)
        issue(b, 0, 0)

        def pair(i2, inner):
            step(b, 2 * i2, 0)
            step(b, 2 * i2 + 1, 1)
            return inner

        lax.fori_loop(0, G // 2, pair, 0)
        step(b, G - 1, 0)
        return carry

    lax.fori_loop(0, NB, block, 0)
    plsc.subcore_barrier()

    for i in range(pl.cdiv(FCHUNKS, NS)):
        cid = s + NS * i

        @pl.when(cid < FCHUNKS)
        def _():
            pltpu.sync_copy(acc.at[pl.ds(cid * FR, FR)],
                            out_hbm.at[c, pl.ds(cid * FR, FR)])


def _make_sc_aggregate():
    mesh = plsc.VectorSubcoreMesh(core_axis_name="c", subcore_axis_name="s",
                                  num_cores=NC, num_subcores=NS)
    return pl.kernel(
        _sc_body,
        out_type=jax.ShapeDtypeStruct((NC, N, D), jnp.float32),
        mesh=mesh,
        scratch_types=[
            pltpu.VMEM((G, CB), jnp.int32),
            pltpu.VMEM((G, CB), jnp.int32),
            pltpu.VMEM((CB, D), jnp.float32),
            pltpu.VMEM((CB, D), jnp.float32),
            pltpu.VMEM((CB, D), jnp.float32),
            pltpu.VMEM((CB, D), jnp.float32),
            pltpu.VMEM_SHARED((N, D), jnp.float32),
            pltpu.SemaphoreType.DMA,
            pltpu.SemaphoreType.DMA,
            pltpu.SemaphoreType.DMA,
            pltpu.SemaphoreType.DMA,
        ],
    )


def _mlp_body(h_ref, a0_ref, a1_ref, w1_ref, b1_ref, w2_ref, b2_ref, o_ref):
    t = h_ref[...] + a0_ref[...] + a1_ref[...]
    u = jnp.maximum(
        jnp.dot(t, w1_ref[...], preferred_element_type=jnp.float32)
        + b1_ref[...], 0.0)
    v = jnp.maximum(
        jnp.dot(u, w2_ref[...], preferred_element_type=jnp.float32)
        + b2_ref[...], 0.0)
    o_ref[...] = v


def _jk_body(h1_ref, h2_ref, h3_ref, wc1_ref, wc2_ref, wc3_ref, bc_ref, o_ref):
    acc = jnp.dot(h1_ref[...], wc1_ref[...], preferred_element_type=jnp.float32)
    acc += jnp.dot(h2_ref[...], wc2_ref[...], preferred_element_type=jnp.float32)
    acc += jnp.dot(h3_ref[...], wc3_ref[...], preferred_element_type=jnp.float32)
    o_ref[...] = jnp.maximum(acc + bc_ref[...], 0.0)


_ROWS = 1000
_GRID = N // _ROWS


def _row_spec():
    return pl.BlockSpec((_ROWS, D), lambda i: (i, 0))


def _full_spec():
    return pl.BlockSpec((D, D), lambda i: (0, 0))


def _bias_spec():
    return pl.BlockSpec((1, D), lambda i: (0, 0))


def _mlp(h, a0, a1, w1, b1, w2, b2):
    return pl.pallas_call(
        _mlp_body,
        grid=(_GRID,),
        in_specs=[_row_spec(), _row_spec(), _row_spec(),
                  _full_spec(), _bias_spec(), _full_spec(), _bias_spec()],
        out_specs=_row_spec(),
        out_shape=jax.ShapeDtypeStruct((N, D), jnp.float32),
    )(h, a0, a1, w1, b1.reshape(1, D), w2, b2.reshape(1, D))


def _jk(h1, h2, h3, wc, bc):
    return pl.pallas_call(
        _jk_body,
        grid=(_GRID,),
        in_specs=[_row_spec(), _row_spec(), _row_spec(),
                  _full_spec(), _full_spec(), _full_spec(), _bias_spec()],
        out_specs=_row_spec(),
        out_shape=jax.ShapeDtypeStruct((N, D), jnp.float32),
    )(h1, h2, h3, wc[:D], wc[D:2 * D], wc[2 * D:], bc.reshape(1, D))


@jax.jit
def kernel(x, edge_index, edge_attr,
           W1_0, b1_0, W2_0, b2_0,
           W1_1, b1_1, W2_1, b2_1,
           W1_2, b1_2, W2_2, b2_2,
           Wc, bc):
    src = edge_index[0].astype(jnp.int32).reshape(NW, NB, G, CB)
    dst = edge_index[1].astype(jnp.int32).reshape(NW, NB, G, CB)
    ea = edge_attr.reshape(SUBS, CB, D)

    aggregate = _make_sc_aggregate()

    params = [(W1_0, b1_0, W2_0, b2_0),
              (W1_1, b1_1, W2_1, b2_1),
              (W1_2, b1_2, W2_2, b2_2)]
    h = x
    xs = []
    for (w1, b1, w2, b2) in params:
        agg = aggregate(h, src, dst, ea)
        h = _mlp(h, agg[0], agg[1], w1, b1, w2, b2)
        xs.append(h)
    return _jk(xs[0], xs[1], xs[2], Wc, bc)


# speedup vs baseline: 6.7118x; 1.0014x over previous
import functools

import jax
import jax.numpy as jnp
from jax import lax
from jax.experimental import pallas as pl
from jax.experimental.pallas import tpu as pltpu
from jax.experimental.pallas import tpu_sc as plsc

N = 10000
E = 320000
D = 128

NC = 2
NS = 16
NW = NC * NS

CB = 80
SUBS = E // CB
PER_W = SUBS // NW
G = 25
NB = PER_W // G
FR = 80
FCHUNKS = N // FR


def _sc_body(h_hbm, src_hbm, dst_hbm, ea_hbm, out_hbm,
             sidx, didx, gbuf0, gbuf1, ebuf0, ebuf1, acc,
             gsem0, gsem1, esem0, esem1, ssem0, ssem1):
    c = lax.axis_index("c")
    s = lax.axis_index("s")
    wid = s * NC + c
    gbufs = (gbuf0, gbuf1)
    ebufs = (ebuf0, ebuf1)
    gsems = (gsem0, gsem1)
    esems = (esem0, esem1)
    ssems = (ssem0, ssem1)

    zero = jnp.zeros((16,), jnp.float32)

    def zrow(r, carry):
        for k in range(D // 16):
            gbuf0[r, pl.ds(k * 16, 16)] = zero
        return carry

    lax.fori_loop(0, FR, zrow, 0)
    for i in range(pl.cdiv(FCHUNKS, NS)):
        cid = s + NS * i

        @pl.when(cid < FCHUNKS)
        def _():
            pltpu.sync_copy(gbuf0.at[pl.ds(0, FR)],
                            acc.at[pl.ds(cid * FR, FR)])

    plsc.subcore_barrier()

    def issue(b, i, p):
        jg = (wid * NB + b) * G + i
        pltpu.async_copy(ea_hbm.at[jg], ebufs[p], esems[p])
        pltpu.async_copy(h_hbm.at[sidx.at[i]], gbufs[p], gsems[p])

    def step(b, i, p):
        q = 1 - p
        gbuf = gbufs[p]
        ebuf = ebufs[p]
        pltpu.make_async_copy(ea_hbm.at[0], ebuf, esems[p]).wait()
        pltpu.make_async_copy(ea_hbm.at[0], gbuf, gsems[p]).wait()

        @pl.when(i + 1 < G)
        def _():
            @pl.when(i > 0)
            def _():
                pltpu.make_async_copy(ea_hbm.at[0], gbufs[q], ssems[q]).wait()

            issue(b, i + 1, q)

        def mrow(r2, inner):
            for u in range(2):
                r = r2 * 2 + u
                for k in range(D // 16):
                    sl = pl.ds(k * 16, 16)
                    gbuf[r, sl] = jnp.maximum(gbuf[r, sl] + ebuf[r, sl], 0.0)
            return inner

        lax.fori_loop(0, CB // 2, mrow, 0)
        pltpu.async_copy(gbuf, acc.at[didx.at[i]], ssems[p], add=True)

    def block(b, carry):
        @pl.when(b > 0)
        def _():
            pltpu.make_async_copy(ea_hbm.at[0], gbufs[1], ssems[1]).wait()
            pltpu.make_async_copy(ea_hbm.at[0], gbufs[0], ssems[0]).wait()

        pltpu.sync_copy(src_hbm.at[wid, b], sidx)
        pltpu.sync_copy(dst_hbm.at[wid, b], didx)
        issue(b, 0, 0)

        def pair(i2, inner):
            step(b, 2 * i2, 0)
            step(b, 2 * i2 + 1, 1)
            return inner

        lax.fori_loop(0, G // 2, pair, 0)
        step(b, G - 1, 0)
        return carry

    lax.fori_loop(0, NB, block, 0)
    pltpu.make_async_copy(ea_hbm.at[0], gbufs[1], ssems[1]).wait()
    pltpu.make_async_copy(ea_hbm.at[0], gbufs[0], ssems[0]).wait()
    plsc.subcore_barrier()

    for i in range(pl.cdiv(FCHUNKS, NS)):
        cid = s + NS * i

        @pl.when(cid < FCHUNKS)
        def _():
            pltpu.sync_copy(acc.at[pl.ds(cid * FR, FR)],
                            out_hbm.at[c, pl.ds(cid * FR, FR)])


def _make_sc_aggregate():
    mesh = plsc.VectorSubcoreMesh(core_axis_name="c", subcore_axis_name="s",
                                  num_cores=NC, num_subcores=NS)
    return pl.kernel(
        _sc_body,
        out_type=jax.ShapeDtypeStruct((NC, N, D), jnp.float32),
        mesh=mesh,
        scratch_types=[
            pltpu.VMEM((G, CB), jnp.int32),
            pltpu.VMEM((G, CB), jnp.int32),
            pltpu.VMEM((CB, D), jnp.float32),
            pltpu.VMEM((CB, D), jnp.float32),
            pltpu.VMEM((CB, D), jnp.float32),
            pltpu.VMEM((CB, D), jnp.float32),
            pltpu.VMEM_SHARED((N, D), jnp.float32),
            pltpu.SemaphoreType.DMA,
            pltpu.SemaphoreType.DMA,
            pltpu.SemaphoreType.DMA,
            pltpu.SemaphoreType.DMA,
            pltpu.SemaphoreType.DMA,
            pltpu.SemaphoreType.DMA,
        ],
    )


def _mlp_body(h_ref, a0_ref, a1_ref, w1_ref, b1_ref, w2_ref, b2_ref, o_ref):
    t = h_ref[...] + a0_ref[...] + a1_ref[...]
    u = jnp.maximum(
        jnp.dot(t, w1_ref[...], preferred_element_type=jnp.float32)
        + b1_ref[...], 0.0)
    v = jnp.maximum(
        jnp.dot(u, w2_ref[...], preferred_element_type=jnp.float32)
        + b2_ref[...], 0.0)
    o_ref[...] = v


def _jk_body(h1_ref, h2_ref, h3_ref, wc1_ref, wc2_ref, wc3_ref, bc_ref, o_ref):
    acc = jnp.dot(h1_ref[...], wc1_ref[...], preferred_element_type=jnp.float32)
    acc += jnp.dot(h2_ref[...], wc2_ref[...], preferred_element_type=jnp.float32)
    acc += jnp.dot(h3_ref[...], wc3_ref[...], preferred_element_type=jnp.float32)
    o_ref[...] = jnp.maximum(acc + bc_ref[...], 0.0)


_ROWS = 1000
_GRID = N // _ROWS


def _row_spec():
    return pl.BlockSpec((_ROWS, D), lambda i: (i, 0))


def _full_spec():
    return pl.BlockSpec((D, D), lambda i: (0, 0))


def _bias_spec():
    return pl.BlockSpec((1, D), lambda i: (0, 0))


def _mlp(h, a0, a1, w1, b1, w2, b2):
    return pl.pallas_call(
        _mlp_body,
        grid=(_GRID,),
        in_specs=[_row_spec(), _row_spec(), _row_spec(),
                  _full_spec(), _bias_spec(), _full_spec(), _bias_spec()],
        out_specs=_row_spec(),
        out_shape=jax.ShapeDtypeStruct((N, D), jnp.float32),
    )(h, a0, a1, w1, b1.reshape(1, D), w2, b2.reshape(1, D))


def _jk(h1, h2, h3, wc, bc):
    return pl.pallas_call(
        _jk_body,
        grid=(_GRID,),
        in_specs=[_row_spec(), _row_spec(), _row_spec(),
                  _full_spec(), _full_spec(), _full_spec(), _bias_spec()],
        out_specs=_row_spec(),
        out_shape=jax.ShapeDtypeStruct((N, D), jnp.float32),
    )(h1, h2, h3, wc[:D], wc[D:2 * D], wc[2 * D:], bc.reshape(1, D))


@jax.jit
def kernel(x, edge_index, edge_attr,
           W1_0, b1_0, W2_0, b2_0,
           W1_1, b1_1, W2_1, b2_1,
           W1_2, b1_2, W2_2, b2_2,
           Wc, bc):
    src = edge_index[0].astype(jnp.int32).reshape(NW, NB, G, CB)
    dst = edge_index[1].astype(jnp.int32).reshape(NW, NB, G, CB)
    ea = edge_attr.reshape(SUBS, CB, D)

    aggregate = _make_sc_aggregate()

    params = [(W1_0, b1_0, W2_0, b2_0),
              (W1_1, b1_1, W2_1, b2_1),
              (W1_2, b1_2, W2_2, b2_2)]
    h = x
    xs = []
    for (w1, b1, w2, b2) in params:
        agg = aggregate(h, src, dst, ea)
        h = _mlp(h, agg[0], agg[1], w1, b1, w2, b2)
        xs.append(h)
    return _jk(xs[0], xs[1], xs[2], Wc, bc)
